# Initial kernel scaffold; baseline (speedup 1.0000x reference)
import jax
import jax.numpy as jnp
from jax.experimental import pallas as pl


def kernel(x_nchw, w_oc_ic, gamma, beta):
    raise NotImplementedError("write your pallas kernel here")


# speedup vs baseline: 1.5008x; 1.5008x over previous
import functools

import jax
import jax.numpy as jnp
from jax.experimental import pallas as pl
from jax.experimental.pallas import tpu as pltpu

_VMEM_LIMIT_BYTES = 48 * 1024 * 1024
_AUG = 48


def _stats_kernel(x_ref, sa_ref, *, cin):
    bn, _, hw = x_ref.shape
    pad = jnp.ones((_AUG - cin, hw), jnp.float32)
    acc = None
    for j in range(bn):
        xa = jnp.concatenate([x_ref[j], pad], axis=0)
        g = jax.lax.dot_general(xa, xa, (((1,), (1,)), ((), ())),
                                preferred_element_type=jnp.float32)
        acc = g if acc is None else acc + g
    sa_ref[...] = acc.reshape(sa_ref.shape)


def _apply_kernel(w_ref, x_ref, sa_ref, gamma_ref, beta_ref, o_ref,
                  *, eps, inv_m):
    cin = w_ref.shape[1]
    sa = sa_ref[0]
    for g in range(1, sa_ref.shape[0]):
        sa = sa + sa_ref[g]
    w = w_ref[...]
    t = jnp.dot(w, sa[:cin, :], preferred_element_type=jnp.float32)
    mean = t[:, cin:cin + 1] * inv_m
    ey2 = jnp.sum(t[:, :cin] * w, axis=1, keepdims=True) * inv_m
    var = jnp.maximum(ey2 - mean * mean, 0.0)
    scale = gamma_ref[...] * jax.lax.rsqrt(var + eps)
    shift = beta_ref[...] - mean * scale

    for j in range(x_ref.shape[0]):
        y = jnp.dot(w, x_ref[j], preferred_element_type=jnp.float32)
        o_ref[j] = y * scale + shift


def _largest_div(n, cands):
    for c in cands:
        if n % c == 0:
            return c
    return 1


@jax.jit
def kernel(x_nchw, w_oc_ic, gamma, beta):
    eps = 1e-3
    n, cin, h, w_sp = x_nchw.shape
    cout = w_oc_ic.shape[0]
    hw = h * w_sp
    m = n * hw

    x3 = x_nchw.reshape(n, cin, hw)
    gamma2 = gamma.reshape(cout, 1).astype(jnp.float32)
    beta2 = beta.reshape(cout, 1).astype(jnp.float32)

    bn_s = _largest_div(n, (8, 4, 2))
    grp = n // bn_s
    sa_part = pl.pallas_call(
        functools.partial(_stats_kernel, cin=cin),
        out_shape=jax.ShapeDtypeStruct((grp, _AUG, _AUG), jnp.float32),
        grid=(grp,),
        in_specs=[pl.BlockSpec((bn_s, cin, hw), lambda i: (i, 0, 0))],
        out_specs=pl.BlockSpec((1, _AUG, _AUG), lambda i: (i, 0, 0)),
        compiler_params=pltpu.CompilerParams(
            dimension_semantics=("parallel",),
            vmem_limit_bytes=_VMEM_LIMIT_BYTES,
        ),
    )(x3)

    bn = _largest_div(n, (2,))
    out = pl.pallas_call(
        functools.partial(_apply_kernel, eps=float(eps), inv_m=1.0 / float(m)),
        out_shape=jax.ShapeDtypeStruct((n, cout, hw), jnp.float32),
        grid=(n // bn,),
        in_specs=[
            pl.BlockSpec((cout, cin), lambda i: (0, 0)),
            pl.BlockSpec((bn, cin, hw), lambda i: (i, 0, 0)),
            pl.BlockSpec((grp, _AUG, _AUG), lambda i: (0, 0, 0)),
            pl.BlockSpec((cout, 1), lambda i: (0, 0)),
            pl.BlockSpec((cout, 1), lambda i: (0, 0)),
        ],
        out_specs=pl.BlockSpec((bn, cout, hw), lambda i: (i, 0, 0)),
        compiler_params=pltpu.CompilerParams(
            dimension_semantics=("parallel",),
            vmem_limit_bytes=_VMEM_LIMIT_BYTES,
        ),
    )(w_oc_ic, x3, sa_part, gamma2, beta2)

    return out.reshape(n, cout, h, w_sp)
